# baseline (speedup 1.0000x reference)
import functools
import math

import jax
import jax.numpy as jnp
from jax import lax
from jax.experimental import pallas as pl
from jax.experimental.pallas import tpu as pltpu
from jax.experimental.pallas import tpu_sc as plsc

_B = 16384
_L = 50
_LP = 52
_D = 64
_C = 3
_V = 1000000
_VP = 1 << 20
_NC = 2
_NS = 16
_NW = _NC * _NS
_BAGS_W = _B // _NW
_BAGS_CHUNK = 2
_IDX_CHUNK = _BAGS_CHUNK * _LP
_CHUNKS = _BAGS_W // _BAGS_CHUNK
_KCH = 8
_GROUPS = _CHUNKS // _KCH
_GBUF = 4
_SCALE = math.sqrt(_D) / _L
_NBLK = 8192
_VW = _VP // _NW
_VCH = 2048
_NCH = _VW // _VCH


def _proj_body(t_ref, w_ref, o0_ref, o1_ref, o2_ref):
    res = lax.dot_general(
        w_ref[...], t_ref[...],
        dimension_numbers=(((1,), (0,)), ((), ())),
        preferred_element_type=jnp.float32)
    o0_ref[...] = res[0]
    o1_ref[...] = res[1]
    o2_ref[...] = res[2]


def _ilv_body(p0_hbm, p1_hbm, p2_hbm, out_hbm,
              in00, in01, in02, in10, in11, in12, blk0, blk1,
              isem0, isem1, osem0, osem1):
    ins = ((in00, in01, in02), (in10, in11, in12))
    blks = (blk0, blk1)
    isems = (isem0, isem1)
    osems = (osem0, osem1)
    p_hbm = (p0_hbm, p1_hbm, p2_hbm)
    wid = lax.axis_index("s") * _NC + lax.axis_index("c")
    base = wid * _VW
    lane = jnp.arange(16, dtype=jnp.int32)

    def in_start(k, s):
        for c in range(_C):
            pltpu.async_copy(p_hbm[c].at[pl.ds(base + k * _VCH, _VCH)],
                             ins[s][c], isems[s])

    def in_wait(s):
        for c in range(_C):
            pltpu.make_async_copy(p_hbm[c].at[pl.ds(0, _VCH)],
                                  ins[s][c], isems[s]).wait()

    def out_desc(k, s):
        return pltpu.make_async_copy(
            blks[s], out_hbm.at[pl.ds(base + k * _VCH, _VCH)], osems[s])

    in_start(0, 0)
    for k in range(_NCH):
        s = k % 2
        in_wait(s)
        if k + 1 < _NCH:
            in_start(k + 1, 1 - s)
        if k >= 2:
            out_desc(k - 2, s).wait()

        def scat(j, carry):
            row = j * 16 + lane
            for c in range(_C):
                plsc.store_scatter(blks[s], [row, jnp.full((16,), c, jnp.int32)],
                                   ins[s][c][pl.ds(j * 16, 16)])
            return carry

        lax.fori_loop(0, _VCH // 16, scat, 0)
        out_desc(k, s).start()
    out_desc(_NCH - 2, 0 if _NCH % 2 == 0 else 1).wait()
    out_desc(_NCH - 1, 1 if _NCH % 2 == 0 else 0).wait()


def _sc_body(texts_hbm, p16_hbm, b_hbm, out_hbm,
             idx_all, b_v, out_v, *ring):
    rows = ring[:_GBUF]
    gsem = ring[_GBUF:]
    wid = lax.axis_index("s") * _NC + lax.axis_index("c")
    bag_base = wid * _BAGS_W

    pltpu.sync_copy(texts_hbm.at[pl.ds(wid * _GROUPS, _GROUPS)], idx_all)
    pltpu.sync_copy(b_hbm, b_v)
    bvec = b_v[pl.ds(0, 16)]
    lane = jnp.arange(16, dtype=jnp.int32)
    lane_ok = lane < _C

    def gather(slot, g):
        return pltpu.make_async_copy(
            p16_hbm.at[idx_all.at[g]],
            rows[slot], gsem[slot])

    for s in range(_GBUF):
        gather(s, s).start()

    def outer(i, carry):
        for s in range(_GBUF):
            g = i * _GBUF + s
            gather(s, g).wait()
            for q in range(_KCH):
                for bb in range(_BAGS_CHUNK):
                    base = q * _IDX_CHUNK + bb * _LP
                    acc = [rows[s][base + k, pl.ds(0, 16)]
                           for k in range(4)]
                    for j in range(4, _L, 4):
                        for k in range(4):
                            if j + k < _L:
                                acc[k] = acc[k] + rows[s][base + j + k,
                                                          pl.ds(0, 16)]
                    bag = (g * _KCH + q) * _BAGS_CHUNK + bb
                    plsc.store_scatter(
                        out_v, [bag * _C + lane],
                        (acc[0] + acc[1]) + (acc[2] + acc[3]) + bvec,
                        mask=lane_ok)

            @pl.when(g + _GBUF < _GROUPS)
            def _():
                gather(s, g + _GBUF).start()
        return carry

    lax.fori_loop(0, _GROUPS // _GBUF, outer, 0)
    pltpu.sync_copy(out_v, out_hbm.at[pl.ds(bag_base * _C, _BAGS_W * _C)])


@jax.jit
def _run(texts2d, table_t, wp, b16):
    p0, p1, p2 = pl.pallas_call(
        _proj_body,
        grid=(-(-_V // _NBLK),),
        in_specs=[
            pl.BlockSpec((_D, _NBLK), lambda j: (0, j)),
            pl.BlockSpec((8, _D), lambda j: (0, 0)),
        ],
        out_specs=[pl.BlockSpec((_NBLK,), lambda j: (j,))] * _C,
        out_shape=[jax.ShapeDtypeStruct((_VP,), jnp.float32)] * _C,
    )(table_t, wp)

    mesh = plsc.VectorSubcoreMesh(core_axis_name="c", subcore_axis_name="s")
    sc_params = pltpu.CompilerParams(
        needs_layout_passes=False, use_tc_tiling_on_sc=False)

    p16 = pl.kernel(
        _ilv_body,
        out_type=jax.ShapeDtypeStruct((_VP, 16), jnp.float32),
        mesh=mesh,
        scratch_types=(
            [pltpu.VMEM((_VCH,), jnp.float32)] * 6
            + [pltpu.VMEM((_VCH, 16), jnp.float32)] * 2
            + [pltpu.SemaphoreType.DMA] * 4
        ),
        compiler_params=sc_params,
    )(p0, p1, p2)

    out = pl.kernel(
        _sc_body,
        out_type=jax.ShapeDtypeStruct((_B * _C,), jnp.float32),
        mesh=mesh,
        scratch_types=(
            [pltpu.VMEM((_GROUPS, _KCH * _IDX_CHUNK), jnp.int32),
             pltpu.VMEM((16,), jnp.float32),
             pltpu.VMEM((_BAGS_W * _C,), jnp.float32)]
            + [pltpu.VMEM((_KCH * _IDX_CHUNK, 16), jnp.float32)] * _GBUF
            + [pltpu.SemaphoreType.DMA] * _GBUF
        ),
        compiler_params=sc_params,
    )(texts2d, p16, b16)
    return out


def kernel(texts, table, W, b):
    texts_p = jnp.pad(texts, ((0, 0), (0, _LP - _L)))
    texts2d = texts_p.reshape(-1, _KCH * _IDX_CHUNK)
    wp = jnp.zeros((8, _D), W.dtype).at[:_C].set(W * _SCALE)
    b16 = jnp.zeros((16,), b.dtype).at[:_C].set(b)
    return _run(texts2d, table.T, wp, b16).reshape(_B, _C)


# speedup vs baseline: 1.1255x; 1.1255x over previous
import functools
import math

import jax
import jax.numpy as jnp
from jax import lax
from jax.experimental import pallas as pl
from jax.experimental.pallas import tpu as pltpu
from jax.experimental.pallas import tpu_sc as plsc

_B = 16384
_L = 50
_LP = 52
_D = 64
_C = 3
_V = 1000000
_VP = 1 << 20
_NC = 2
_NS = 16
_NW = _NC * _NS
_BAGS_W = _B // _NW
_BAGS_CHUNK = 2
_IDX_CHUNK = _BAGS_CHUNK * _LP
_CHUNKS = _BAGS_W // _BAGS_CHUNK
_KCH = 1
_GROUPS = _CHUNKS // _KCH
_GBUF = 8
_SCALE = math.sqrt(_D) / _L
_NBLK = 16384
_VW = _VP // _NW
_VCH = 2048
_NCH = _VW // _VCH


def _proj_body(t_ref, w_ref, o0_ref, o1_ref, o2_ref):
    res = lax.dot_general(
        w_ref[...], t_ref[...],
        dimension_numbers=(((1,), (0,)), ((), ())),
        preferred_element_type=jnp.float32)
    o0_ref[...] = res[0]
    o1_ref[...] = res[1]
    o2_ref[...] = res[2]


def _ilv_body(p0_hbm, p1_hbm, p2_hbm, out_hbm,
              in00, in01, in02, in10, in11, in12, blk0, blk1,
              isem0, isem1, osem0, osem1):
    ins = ((in00, in01, in02), (in10, in11, in12))
    blks = (blk0, blk1)
    isems = (isem0, isem1)
    osems = (osem0, osem1)
    p_hbm = (p0_hbm, p1_hbm, p2_hbm)
    wid = lax.axis_index("s") * _NC + lax.axis_index("c")
    base = wid * _VW
    lane = jnp.arange(16, dtype=jnp.int32)

    def in_start(k, s):
        for c in range(_C):
            pltpu.async_copy(p_hbm[c].at[pl.ds(base + k * _VCH, _VCH)],
                             ins[s][c], isems[s])

    def in_wait(s):
        for c in range(_C):
            pltpu.make_async_copy(p_hbm[c].at[pl.ds(0, _VCH)],
                                  ins[s][c], isems[s]).wait()

    def out_desc(k, s):
        return pltpu.make_async_copy(
            blks[s], out_hbm.at[pl.ds(base + k * _VCH, _VCH)], osems[s])

    in_start(0, 0)
    for k in range(_NCH):
        s = k % 2
        in_wait(s)
        if k + 1 < _NCH:
            in_start(k + 1, 1 - s)
        if k >= 2:
            out_desc(k - 2, s).wait()

        def scat(j, carry):
            row = j * 16 + lane
            for c in range(_C):
                plsc.store_scatter(blks[s], [row, jnp.full((16,), c, jnp.int32)],
                                   ins[s][c][pl.ds(j * 16, 16)])
            return carry

        lax.fori_loop(0, _VCH // 16, scat, 0)
        out_desc(k, s).start()
    out_desc(_NCH - 2, 0 if _NCH % 2 == 0 else 1).wait()
    out_desc(_NCH - 1, 1 if _NCH % 2 == 0 else 0).wait()


def _sc_body(texts_hbm, p16_hbm, b_hbm, out_hbm,
             idx_all, b_v, out_v, *ring):
    rows = ring[:_GBUF]
    gsem = ring[_GBUF:]
    wid = lax.axis_index("s") * _NC + lax.axis_index("c")
    bag_base = wid * _BAGS_W

    pltpu.sync_copy(texts_hbm.at[pl.ds(wid * _GROUPS, _GROUPS)], idx_all)
    pltpu.sync_copy(b_hbm, b_v)
    bvec = b_v[pl.ds(0, 16)]
    lane = jnp.arange(16, dtype=jnp.int32)
    lane_ok = lane < _C

    def gather(slot, g):
        return pltpu.make_async_copy(
            p16_hbm.at[idx_all.at[g]],
            rows[slot], gsem[slot])

    for s in range(_GBUF):
        gather(s, s).start()

    def outer(i, carry):
        for s in range(_GBUF):
            g = i * _GBUF + s
            gather(s, g).wait()
            for q in range(_KCH):
                for bb in range(_BAGS_CHUNK):
                    base = q * _IDX_CHUNK + bb * _LP
                    acc = [rows[s][base + k, pl.ds(0, 16)]
                           for k in range(4)]
                    for j in range(4, _L, 4):
                        for k in range(4):
                            if j + k < _L:
                                acc[k] = acc[k] + rows[s][base + j + k,
                                                          pl.ds(0, 16)]
                    bag = (g * _KCH + q) * _BAGS_CHUNK + bb
                    plsc.store_scatter(
                        out_v, [bag * _C + lane],
                        (acc[0] + acc[1]) + (acc[2] + acc[3]) + bvec,
                        mask=lane_ok)

            @pl.when(g + _GBUF < _GROUPS)
            def _():
                gather(s, g + _GBUF).start()
        return carry

    lax.fori_loop(0, _GROUPS // _GBUF, outer, 0)
    pltpu.sync_copy(out_v, out_hbm.at[pl.ds(bag_base * _C, _BAGS_W * _C)])


@jax.jit
def _run(texts2d, table_t, wp, b16):
    p0, p1, p2 = pl.pallas_call(
        _proj_body,
        grid=(-(-_V // _NBLK),),
        in_specs=[
            pl.BlockSpec((_D, _NBLK), lambda j: (0, j)),
            pl.BlockSpec((8, _D), lambda j: (0, 0)),
        ],
        out_specs=[pl.BlockSpec((_NBLK,), lambda j: (j,))] * _C,
        out_shape=[jax.ShapeDtypeStruct((_VP,), jnp.float32)] * _C,
    )(table_t, wp)

    mesh = plsc.VectorSubcoreMesh(core_axis_name="c", subcore_axis_name="s")
    sc_params = pltpu.CompilerParams(
        needs_layout_passes=False, use_tc_tiling_on_sc=False)

    p16 = pl.kernel(
        _ilv_body,
        out_type=jax.ShapeDtypeStruct((_VP, 16), jnp.float32),
        mesh=mesh,
        scratch_types=(
            [pltpu.VMEM((_VCH,), jnp.float32)] * 6
            + [pltpu.VMEM((_VCH, 16), jnp.float32)] * 2
            + [pltpu.SemaphoreType.DMA] * 4
        ),
        compiler_params=sc_params,
    )(p0, p1, p2)

    out = pl.kernel(
        _sc_body,
        out_type=jax.ShapeDtypeStruct((_B * _C,), jnp.float32),
        mesh=mesh,
        scratch_types=(
            [pltpu.VMEM((_GROUPS, _KCH * _IDX_CHUNK), jnp.int32),
             pltpu.VMEM((16,), jnp.float32),
             pltpu.VMEM((_BAGS_W * _C,), jnp.float32)]
            + [pltpu.VMEM((_KCH * _IDX_CHUNK, 16), jnp.float32)] * _GBUF
            + [pltpu.SemaphoreType.DMA] * _GBUF
        ),
        compiler_params=sc_params,
    )(texts2d, p16, b16)
    return out


def kernel(texts, table, W, b):
    texts_p = jnp.pad(texts, ((0, 0), (0, _LP - _L)))
    texts2d = texts_p.reshape(-1, _KCH * _IDX_CHUNK)
    wp = jnp.zeros((8, _D), W.dtype).at[:_C].set(W * _SCALE)
    b16 = jnp.zeros((16,), b.dtype).at[:_C].set(b)
    return _run(texts2d, table.T, wp, b16).reshape(_B, _C)


# speedup vs baseline: 1.1455x; 1.0178x over previous
import functools
import math

import jax
import jax.numpy as jnp
from jax import lax
from jax.experimental import pallas as pl
from jax.experimental.pallas import tpu as pltpu
from jax.experimental.pallas import tpu_sc as plsc

_B = 16384
_L = 50
_LP = 52
_D = 64
_C = 3
_V = 1000000
_VP = 1 << 20
_NC = 2
_NS = 16
_NW = _NC * _NS
_BAGS_W = _B // _NW
_BAGS_CHUNK = 2
_IDX_CHUNK = _BAGS_CHUNK * _LP
_CHUNKS = _BAGS_W // _BAGS_CHUNK
_KCH = 1
_GROUPS = _CHUNKS // _KCH
_GBUF = 16
_SCALE = math.sqrt(_D) / _L
_NBLK = 32768
_VW = _VP // _NW
_VCH = 2048
_NCH = _VW // _VCH


def _proj_body(t_ref, w_ref, o0_ref, o1_ref, o2_ref):
    res = lax.dot_general(
        w_ref[...], t_ref[...],
        dimension_numbers=(((1,), (0,)), ((), ())),
        preferred_element_type=jnp.float32)
    o0_ref[...] = res[0]
    o1_ref[...] = res[1]
    o2_ref[...] = res[2]


def _ilv_body(p0_hbm, p1_hbm, p2_hbm, out_hbm,
              in00, in01, in02, in10, in11, in12, blk0, blk1,
              isem0, isem1, osem0, osem1):
    ins = ((in00, in01, in02), (in10, in11, in12))
    blks = (blk0, blk1)
    isems = (isem0, isem1)
    osems = (osem0, osem1)
    p_hbm = (p0_hbm, p1_hbm, p2_hbm)
    wid = lax.axis_index("s") * _NC + lax.axis_index("c")
    base = wid * _VW
    lane = jnp.arange(16, dtype=jnp.int32)

    def in_start(k, s):
        for c in range(_C):
            pltpu.async_copy(p_hbm[c].at[pl.ds(base + k * _VCH, _VCH)],
                             ins[s][c], isems[s])

    def in_wait(s):
        for c in range(_C):
            pltpu.make_async_copy(p_hbm[c].at[pl.ds(0, _VCH)],
                                  ins[s][c], isems[s]).wait()

    def out_desc(k, s):
        return pltpu.make_async_copy(
            blks[s], out_hbm.at[pl.ds(base + k * _VCH, _VCH)], osems[s])

    in_start(0, 0)
    for k in range(_NCH):
        s = k % 2
        in_wait(s)
        if k + 1 < _NCH:
            in_start(k + 1, 1 - s)
        if k >= 2:
            out_desc(k - 2, s).wait()

        def scat(j, carry):
            row = j * 16 + lane
            for c in range(_C):
                plsc.store_scatter(blks[s], [row, jnp.full((16,), c, jnp.int32)],
                                   ins[s][c][pl.ds(j * 16, 16)])
            return carry

        lax.fori_loop(0, _VCH // 16, scat, 0)
        out_desc(k, s).start()
    out_desc(_NCH - 2, 0 if _NCH % 2 == 0 else 1).wait()
    out_desc(_NCH - 1, 1 if _NCH % 2 == 0 else 0).wait()


def _sc_body(texts_hbm, p16_hbm, b_hbm, out_hbm,
             idx_all, b_v, out_v, *ring):
    rows = ring[:_GBUF]
    gsem = ring[_GBUF:]
    wid = lax.axis_index("s") * _NC + lax.axis_index("c")
    bag_base = wid * _BAGS_W

    pltpu.sync_copy(texts_hbm.at[pl.ds(wid * _GROUPS, _GROUPS)], idx_all)
    pltpu.sync_copy(b_hbm, b_v)
    bvec = b_v[pl.ds(0, 16)]
    lane = jnp.arange(16, dtype=jnp.int32)
    lane_ok = lane < _C

    def gather(slot, g):
        return pltpu.make_async_copy(
            p16_hbm.at[idx_all.at[g]],
            rows[slot], gsem[slot])

    for s in range(_GBUF):
        gather(s, s).start()

    def outer(i, carry):
        for s in range(_GBUF):
            g = i * _GBUF + s
            gather(s, g).wait()
            for q in range(_KCH):
                for bb in range(_BAGS_CHUNK):
                    base = q * _IDX_CHUNK + bb * _LP
                    acc = [rows[s][base + k, pl.ds(0, 16)]
                           for k in range(4)]
                    for j in range(4, _L, 4):
                        for k in range(4):
                            if j + k < _L:
                                acc[k] = acc[k] + rows[s][base + j + k,
                                                          pl.ds(0, 16)]
                    bag = (g * _KCH + q) * _BAGS_CHUNK + bb
                    plsc.store_scatter(
                        out_v, [bag * _C + lane],
                        (acc[0] + acc[1]) + (acc[2] + acc[3]) + bvec,
                        mask=lane_ok)

            @pl.when(g + _GBUF < _GROUPS)
            def _():
                gather(s, g + _GBUF).start()
        return carry

    lax.fori_loop(0, _GROUPS // _GBUF, outer, 0)
    pltpu.sync_copy(out_v, out_hbm.at[pl.ds(bag_base * _C, _BAGS_W * _C)])


@jax.jit
def _run(texts2d, table_t, wp, b16):
    p0, p1, p2 = pl.pallas_call(
        _proj_body,
        grid=(-(-_V // _NBLK),),
        in_specs=[
            pl.BlockSpec((_D, _NBLK), lambda j: (0, j)),
            pl.BlockSpec((8, _D), lambda j: (0, 0)),
        ],
        out_specs=[pl.BlockSpec((_NBLK,), lambda j: (j,))] * _C,
        out_shape=[jax.ShapeDtypeStruct((_VP,), jnp.float32)] * _C,
    )(table_t, wp)

    mesh = plsc.VectorSubcoreMesh(core_axis_name="c", subcore_axis_name="s")
    sc_params = pltpu.CompilerParams(
        needs_layout_passes=False, use_tc_tiling_on_sc=False)

    p16 = pl.kernel(
        _ilv_body,
        out_type=jax.ShapeDtypeStruct((_VP, 16), jnp.float32),
        mesh=mesh,
        scratch_types=(
            [pltpu.VMEM((_VCH,), jnp.float32)] * 6
            + [pltpu.VMEM((_VCH, 16), jnp.float32)] * 2
            + [pltpu.SemaphoreType.DMA] * 4
        ),
        compiler_params=sc_params,
    )(p0, p1, p2)

    out = pl.kernel(
        _sc_body,
        out_type=jax.ShapeDtypeStruct((_B * _C,), jnp.float32),
        mesh=mesh,
        scratch_types=(
            [pltpu.VMEM((_GROUPS, _KCH * _IDX_CHUNK), jnp.int32),
             pltpu.VMEM((16,), jnp.float32),
             pltpu.VMEM((_BAGS_W * _C,), jnp.float32)]
            + [pltpu.VMEM((_KCH * _IDX_CHUNK, 16), jnp.float32)] * _GBUF
            + [pltpu.SemaphoreType.DMA] * _GBUF
        ),
        compiler_params=sc_params,
    )(texts2d, p16, b16)
    return out


def kernel(texts, table, W, b):
    texts_p = jnp.pad(texts, ((0, 0), (0, _LP - _L)))
    texts2d = texts_p.reshape(-1, _KCH * _IDX_CHUNK)
    wp = jnp.zeros((8, _D), W.dtype).at[:_C].set(W * _SCALE)
    b16 = jnp.zeros((16,), b.dtype).at[:_C].set(b)
    return _run(texts2d, table.T, wp, b16).reshape(_B, _C)


# speedup vs baseline: 1.9119x; 1.6690x over previous
import functools
import math

import jax
import jax.numpy as jnp
from jax import lax
from jax.experimental import pallas as pl
from jax.experimental.pallas import tpu as pltpu
from jax.experimental.pallas import tpu_sc as plsc

_B = 16384
_L = 50
_LP = 50
_D = 64
_C = 3
_V = 1000000
_VP = 1 << 20
_NC = 2
_NS = 16
_NW = _NC * _NS
_BAGS_W = _B // _NW
_BAGS_CHUNK = 2
_IDX_CHUNK = _BAGS_CHUNK * _LP
_CHUNKS = _BAGS_W // _BAGS_CHUNK
_KCH = 1
_GROUPS = _CHUNKS // _KCH
_GBUF = 16
_SCALE = math.sqrt(_D) / _L
_NBLK = 32768
_VW = _VP // _NW
_VCH = 2048
_NCH = _VW // _VCH


def _proj_body(t_ref, w_ref, o0_ref, o1_ref, o2_ref):
    res = lax.dot_general(
        w_ref[...], t_ref[...],
        dimension_numbers=(((1,), (0,)), ((), ())),
        preferred_element_type=jnp.float32)
    o0_ref[...] = res[0]
    o1_ref[...] = res[1]
    o2_ref[...] = res[2]


def _ilv_body(p0_hbm, p1_hbm, p2_hbm, out_hbm,
              in00, in01, in02, in10, in11, in12, blk0, blk1,
              isem0, isem1, osem0, osem1):
    ins = ((in00, in01, in02), (in10, in11, in12))
    blks = (blk0, blk1)
    isems = (isem0, isem1)
    osems = (osem0, osem1)
    p_hbm = (p0_hbm, p1_hbm, p2_hbm)
    wid = lax.axis_index("s") * _NC + lax.axis_index("c")
    base = wid * _VW
    lane = jnp.arange(16, dtype=jnp.int32)

    def in_start(k, s):
        for c in range(_C):
            pltpu.async_copy(p_hbm[c].at[pl.ds(base + k * _VCH, _VCH)],
                             ins[s][c], isems[s])

    def in_wait(s):
        for c in range(_C):
            pltpu.make_async_copy(p_hbm[c].at[pl.ds(0, _VCH)],
                                  ins[s][c], isems[s]).wait()

    def out_desc(k, s):
        return pltpu.make_async_copy(
            blks[s], out_hbm.at[pl.ds(base + k * _VCH, _VCH)], osems[s])

    in_start(0, 0)
    for k in range(_NCH):
        s = k % 2
        in_wait(s)
        if k + 1 < _NCH:
            in_start(k + 1, 1 - s)
        if k >= 2:
            out_desc(k - 2, s).wait()

        def scat(j, carry):
            row = j * 16 + lane
            for c in range(_C):
                plsc.store_scatter(blks[s], [row, jnp.full((16,), c, jnp.int32)],
                                   ins[s][c][pl.ds(j * 16, 16)])
            return carry

        lax.fori_loop(0, _VCH // 16, scat, 0)
        out_desc(k, s).start()
    out_desc(_NCH - 2, 0 if _NCH % 2 == 0 else 1).wait()
    out_desc(_NCH - 1, 1 if _NCH % 2 == 0 else 0).wait()


def _sc_body(texts_hbm, p16_hbm, b_hbm, out_hbm,
             idx_all, b_v, out_v, *ring):
    rows = ring[:_GBUF]
    gsem = ring[_GBUF:]
    wid = lax.axis_index("s") * _NC + lax.axis_index("c")
    bag_base = wid * _BAGS_W

    pltpu.sync_copy(texts_hbm.at[pl.ds(wid * _GROUPS, _GROUPS)], idx_all)
    pltpu.sync_copy(b_hbm, b_v)
    bvec = b_v[pl.ds(0, 16)]
    lane = jnp.arange(16, dtype=jnp.int32)
    lane_ok = lane < _C

    def gather(slot, g):
        return pltpu.make_async_copy(
            p16_hbm.at[idx_all.at[g]],
            rows[slot], gsem[slot])

    for s in range(_GBUF):
        gather(s, s).start()

    def outer(i, carry):
        for s in range(_GBUF):
            g = i * _GBUF + s
            gather(s, g).wait()
            for q in range(_KCH):
                for bb in range(_BAGS_CHUNK):
                    base = q * _IDX_CHUNK + bb * _LP
                    acc = [rows[s][base + k, pl.ds(0, 16)]
                           for k in range(4)]
                    for j in range(4, _L, 4):
                        for k in range(4):
                            if j + k < _L:
                                acc[k] = acc[k] + rows[s][base + j + k,
                                                          pl.ds(0, 16)]
                    bag = (g * _KCH + q) * _BAGS_CHUNK + bb
                    plsc.store_scatter(
                        out_v, [bag * _C + lane],
                        (acc[0] + acc[1]) + (acc[2] + acc[3]) + bvec,
                        mask=lane_ok)

            @pl.when(g + _GBUF < _GROUPS)
            def _():
                gather(s, g + _GBUF).start()
        return carry

    lax.fori_loop(0, _GROUPS // _GBUF, outer, 0)
    pltpu.sync_copy(out_v, out_hbm.at[pl.ds(bag_base * _C, _BAGS_W * _C)])


@jax.jit
def _run(texts2d, table_t, wp, b16):
    p0, p1, p2 = pl.pallas_call(
        _proj_body,
        grid=(-(-_V // _NBLK),),
        in_specs=[
            pl.BlockSpec((_D, _NBLK), lambda j: (0, j)),
            pl.BlockSpec((8, _D), lambda j: (0, 0)),
        ],
        out_specs=[pl.BlockSpec((_NBLK,), lambda j: (j,))] * _C,
        out_shape=[jax.ShapeDtypeStruct((_VP,), jnp.float32)] * _C,
    )(table_t, wp)

    mesh = plsc.VectorSubcoreMesh(core_axis_name="c", subcore_axis_name="s")
    sc_params = pltpu.CompilerParams(
        needs_layout_passes=False, use_tc_tiling_on_sc=False)

    p16 = pl.kernel(
        _ilv_body,
        out_type=jax.ShapeDtypeStruct((_VP, 16), jnp.float32),
        mesh=mesh,
        scratch_types=(
            [pltpu.VMEM((_VCH,), jnp.float32)] * 6
            + [pltpu.VMEM((_VCH, 16), jnp.float32)] * 2
            + [pltpu.SemaphoreType.DMA] * 4
        ),
        compiler_params=sc_params,
    )(p0, p1, p2)

    out = pl.kernel(
        _sc_body,
        out_type=jax.ShapeDtypeStruct((_B * _C,), jnp.float32),
        mesh=mesh,
        scratch_types=(
            [pltpu.VMEM((_GROUPS, _KCH * _IDX_CHUNK), jnp.int32),
             pltpu.VMEM((16,), jnp.float32),
             pltpu.VMEM((_BAGS_W * _C,), jnp.float32)]
            + [pltpu.VMEM((_KCH * _IDX_CHUNK, 16), jnp.float32)] * _GBUF
            + [pltpu.SemaphoreType.DMA] * _GBUF
        ),
        compiler_params=sc_params,
    )(texts2d, p16, b16)
    return out


def kernel(texts, table, W, b):
    texts2d = texts.reshape(-1, _KCH * _IDX_CHUNK)
    wp = jnp.zeros((8, _D), W.dtype).at[:_C].set(W * _SCALE)
    b16 = jnp.zeros((16,), b.dtype).at[:_C].set(b)
    return _run(texts2d, table.T, wp, b16).reshape(_B, _C)
